# baseline (speedup 1.0000x reference)
import functools

import jax
import jax.numpy as jnp
import numpy as np
from jax import lax
from jax.experimental import pallas as pl
from jax.experimental.pallas import tpu as pltpu
from jax.experimental.pallas import tpu_sc as plsc

N = 10000
E = 320000
D = 128
D2 = 256
EPS = 1e-5

NC = 2
NS = 16
NW = NC * NS
EPW = E // NW
GC = 80
N_GCHUNK = EPW // GC
NG = D2 // 32
DW = D2 // 2

E_PER_CORE = E // NC
EPT = E_PER_CORE // NS
SC2 = 80
N_SCHUNK = EPT // SC2


def _pack_halves(x):
    half = x.shape[-1] // 2
    lo = jax.lax.bitcast_convert_type(
        x[:, :half].astype(jnp.bfloat16), jnp.uint16
    ).astype(jnp.int32)
    hi = jax.lax.bitcast_convert_type(
        x[:, half:].astype(jnp.bfloat16), jnp.uint16
    ).astype(jnp.int32)
    return lax.bitwise_or(lo, lax.shift_left(hi, 16))


def _node_proj_body(nf_ref, w_ref, b_ref, hs_ref, hd_ref):
    nf = nf_ref[...]
    w = w_ref[...]
    b = b_ref[...]
    hs_ref[...] = _pack_halves(
        jnp.dot(nf, w[:, :D2], preferred_element_type=jnp.float32) + b[:, :D2]
    )
    hd_ref[...] = _pack_halves(
        jnp.dot(nf, w[:, D2:], preferred_element_type=jnp.float32) + b[:, D2:]
    )


def _edge_proj_body(f_ref, w_ref, b_ref, out_ref):
    out_ref[...] = _pack_halves(
        jnp.dot(f_ref[...], w_ref[...], preferred_element_type=jnp.float32)
        + b_ref[...]
    )


def _gate_body(m_ref, stats_ref, gm_ref, bm_ref, msg_ref):
    stats = stats_ref[...]
    ssum = jnp.sum(stats[:, 0, :], axis=0)
    ssq = jnp.sum(stats[:, 1, :], axis=0)
    mean = ssum / E
    var = ssq / E - mean * mean
    rstd = lax.rsqrt(var + EPS)
    scale = rstd * gm_ref[0]
    shift = bm_ref[0] - mean * scale
    mhat = m_ref[...] * scale + shift
    h_f = mhat[:, :D]
    h_s = mhat[:, D:]
    msg_ref[...] = jax.nn.sigmoid(h_f) * jax.nn.softplus(h_s)


def _final_body(nf_ref, hp_ref, gn_ref, bn_ref, out_ref):
    h = hp_ref[0] + hp_ref[1]
    mean = jnp.mean(h, axis=0, keepdims=True)
    var = jnp.mean((h - mean) ** 2, axis=0, keepdims=True)
    rstd = lax.rsqrt(var + EPS)
    hn = (h - mean) * rstd * gn_ref[0] + bn_ref[0]
    out_ref[...] = jax.nn.softplus(nf_ref[...] + hn)


def _sc_mesh():
    return plsc.VectorSubcoreMesh(
        core_axis_name="c", subcore_axis_name="s", num_cores=NC, num_subcores=NS
    )


def _gather_pass(hs32, hd32, src, dst, ep32):
    @functools.partial(
        pl.kernel,
        out_type=[
            jax.ShapeDtypeStruct((E, D2), jnp.float32),
            jax.ShapeDtypeStruct((NW, 2, D2), jnp.float32),
        ],
        mesh=_sc_mesh(),
        scratch_types=[
            pltpu.VMEM((2, GC), jnp.int32),
            pltpu.VMEM((2, GC), jnp.int32),
            pltpu.VMEM((GC, DW), jnp.int32),
            pltpu.VMEM((GC, DW), jnp.int32),
            pltpu.VMEM((GC, DW), jnp.int32),
            pltpu.VMEM((GC, D2), jnp.float32),
            pltpu.VMEM((GC, DW), jnp.int32),
            pltpu.VMEM((GC, DW), jnp.int32),
            pltpu.VMEM((GC, DW), jnp.int32),
            pltpu.VMEM((GC, D2), jnp.float32),
            pltpu.VMEM((D2,), jnp.float32),
            pltpu.VMEM((D2,), jnp.float32),
        ]
        + [pltpu.SemaphoreType.DMA] * 8,
    )
    def k(hs_hbm, hd_hbm, src_hbm, dst_hbm, ep_hbm, m_hbm, stats_hbm,
          ia, ib, ra0, rb0, rc0, rm0, ra1, rb1, rc1, rm1, accs, accq,
          sa0, sb0, sc0, sw0, sa1, sb1, sc1, sw1):
        cid = lax.axis_index("c")
        sid = lax.axis_index("s")
        wid = sid * NC + cid
        base0 = wid * EPW

        bufs = ((ra0, rb0, rc0, rm0, sa0, sb0, sc0, sw0),
                (ra1, rb1, rc1, rm1, sa1, sb1, sc1, sw1))

        zero = jnp.zeros((16,), jnp.float32)
        for g in range(D2 // 16):
            accs[pl.ds(g * 16, 16)] = zero
            accq[pl.ds(g * 16, 16)] = zero

        def issue(ch, b, drain):
            ra, rb, rc, rm, sa, sb, sc_, sw = bufs[b]
            if drain is not None:
                @pl.when(drain)
                def _():
                    pltpu.make_async_copy(
                        rm, m_hbm.at[pl.ds(pl.multiple_of(base0, 8), GC)], sw
                    ).wait()
            cb = pl.multiple_of(base0 + ch * GC, 8)
            pltpu.sync_copy(src_hbm.at[pl.ds(cb, GC)], ia.at[b])
            pltpu.sync_copy(dst_hbm.at[pl.ds(cb, GC)], ib.at[b])
            pltpu.async_copy(hs_hbm.at[ia.at[b]], ra, sa)
            pltpu.async_copy(hd_hbm.at[ib.at[b]], rb, sb)
            pltpu.async_copy(ep_hbm.at[pl.ds(cb, GC)], rc, sc_)

        hi_mask = jnp.full((16,), -65536, jnp.int32)

        def widen(u):
            ev = lax.bitcast_convert_type(lax.shift_left(u, 16), jnp.float32)
            od = lax.bitcast_convert_type(
                lax.bitwise_and(u, hi_mask), jnp.float32
            )
            return ev, od

        def compute(ch, b):
            ra, rb, rc, rm, sa, sb, sc_, sw = bufs[b]
            lin = ep_hbm.at[pl.ds(pl.multiple_of(base0, 8), GC)]
            pltpu.make_async_copy(lin, ra, sa).wait()
            pltpu.make_async_copy(lin, rb, sb).wait()
            pltpu.make_async_copy(lin, rc, sc_).wait()

            init = (jnp.zeros((16,), jnp.float32),) * (4 * NG)

            @plsc.parallel_loop(0, GC, 1, unroll=4, carry=init)
            def sums(r, carry):
                cs = list(carry)
                for g in range(NG):
                    slw = pl.ds(g * 16, 16)
                    ae, ao = widen(ra[r, slw])
                    be, bo = widen(rb[r, slw])
                    ce, co = widen(rc[r, slw])
                    ve = ae + be + ce
                    vo = ao + bo + co
                    rm[r, pl.ds(g * 16, 16)] = ve
                    rm[r, pl.ds(D + g * 16, 16)] = vo
                    cs[2 * g] = cs[2 * g] + ve
                    cs[2 * g + 1] = cs[2 * g + 1] + vo
                    cs[2 * NG + 2 * g] = cs[2 * NG + 2 * g] + ve * ve
                    cs[2 * NG + 2 * g + 1] = cs[2 * NG + 2 * g + 1] + vo * vo
                return tuple(cs)
            for g in range(NG):
                plsc.addupdate(accs.at[pl.ds(g * 16, 16)], sums[2 * g])
                plsc.addupdate(accs.at[pl.ds(D + g * 16, 16)], sums[2 * g + 1])
                plsc.addupdate(accq.at[pl.ds(g * 16, 16)], sums[2 * NG + 2 * g])
                plsc.addupdate(
                    accq.at[pl.ds(D + g * 16, 16)], sums[2 * NG + 2 * g + 1]
                )
            cb = pl.multiple_of(base0 + ch * GC, 8)
            pltpu.async_copy(rm, m_hbm.at[pl.ds(cb, GC)], sw)

        issue(0, 0, drain=None)

        def body(i, carry):
            @pl.when(i % 2 == 0)
            def _():
                issue(i + 1, 1, drain=i >= 1)
                compute(i, 0)

            @pl.when(i % 2 == 1)
            def _():
                issue(i + 1, 0, drain=i >= 1)
                compute(i, 1)

            return carry

        lax.fori_loop(0, N_GCHUNK - 1, body, 0)
        compute(N_GCHUNK - 1, (N_GCHUNK - 1) % 2)

        for b in (0, 1):
            rm = bufs[b][3]
            sw = bufs[b][7]
            pltpu.make_async_copy(
                rm, m_hbm.at[pl.ds(pl.multiple_of(base0, 8), GC)], sw
            ).wait()
        pltpu.sync_copy(accs, stats_hbm.at[wid, 0])
        pltpu.sync_copy(accq, stats_hbm.at[wid, 1])

    return k(hs32, hd32, src, dst, ep32)


def _scatter_pass(msg, dst, zero_init):
    @functools.partial(
        pl.kernel,
        out_type=jax.ShapeDtypeStruct((NC, N, D), jnp.float32),
        mesh=_sc_mesh(),
        scratch_types=[
            pltpu.VMEM((SC2,), jnp.int32),
            pltpu.VMEM((SC2, D), jnp.float32),
            pltpu.VMEM_SHARED((N, D), jnp.float32),
        ],
    )
    def k(msg_hbm, dst_hbm, zero_hbm, out_hbm, idx_v, buf, acc_sh):
        cid = lax.axis_index("c")
        sid = lax.axis_index("s")

        @pl.when(sid == 0)
        def _():
            pltpu.sync_copy(zero_hbm, acc_sh)

        plsc.subcore_barrier()

        base0 = cid * E_PER_CORE + sid * EPT

        def chunk_body(ch, carry):
            cb = pl.multiple_of(base0 + ch * SC2, 8)
            pltpu.sync_copy(dst_hbm.at[pl.ds(cb, SC2)], idx_v)
            pltpu.sync_copy(msg_hbm.at[pl.ds(cb, SC2)], buf)
            pltpu.sync_copy(buf, acc_sh.at[idx_v], add=True)
            return carry

        lax.fori_loop(0, N_SCHUNK, chunk_body, 0)
        plsc.subcore_barrier()
        rb = sid * 624

        @pl.when(sid < NS - 1)
        def _():
            pltpu.sync_copy(
                acc_sh.at[pl.ds(rb, 624)], out_hbm.at[cid, pl.ds(rb, 624)]
            )

        @pl.when(sid == NS - 1)
        def _():
            pltpu.sync_copy(
                acc_sh.at[pl.ds(15 * 624, N - 15 * 624)],
                out_hbm.at[cid, pl.ds(15 * 624, N - 15 * 624)],
            )

    return k(msg, dst, zero_init)


def kernel(node_feats, edge_index, edge_feats, W_src, b_src, W_dst, b_dst,
           W_edge, b_edge, gamma_m, beta_m, gamma_n, beta_n):
    src = edge_index[0].astype(jnp.int32)
    dst = edge_index[1].astype(jnp.int32)

    w_cat = jnp.concatenate([W_src.T, W_dst.T], axis=1)
    b_cat = jnp.concatenate([b_src, b_dst]).reshape(1, 2 * D2)

    hs32, hd32 = pl.pallas_call(
        _node_proj_body,
        out_shape=[
            jax.ShapeDtypeStruct((N, DW), jnp.int32),
            jax.ShapeDtypeStruct((N, DW), jnp.int32),
        ],
    )(node_feats, w_cat, b_cat)

    EB = 4000
    ep32 = pl.pallas_call(
        _edge_proj_body,
        grid=(E // EB,),
        in_specs=[
            pl.BlockSpec((EB, 16), lambda i: (i, 0)),
            pl.BlockSpec((16, D2), lambda i: (0, 0)),
            pl.BlockSpec((1, D2), lambda i: (0, 0)),
        ],
        out_specs=pl.BlockSpec((EB, DW), lambda i: (i, 0)),
        out_shape=jax.ShapeDtypeStruct((E, DW), jnp.int32),
    )(edge_feats, W_edge.T, b_edge.reshape(1, D2))

    m, stats = _gather_pass(hs32, hd32, src, dst, ep32)

    msg = pl.pallas_call(
        _gate_body,
        grid=(E // EB,),
        in_specs=[
            pl.BlockSpec((EB, D2), lambda i: (i, 0)),
            pl.BlockSpec((NW, 2, D2), lambda i: (0, 0, 0)),
            pl.BlockSpec((1, D2), lambda i: (0, 0)),
            pl.BlockSpec((1, D2), lambda i: (0, 0)),
        ],
        out_specs=pl.BlockSpec((EB, D), lambda i: (i, 0)),
        out_shape=jax.ShapeDtypeStruct((E, D), jnp.float32),
    )(m, stats, gamma_m.reshape(1, D2), beta_m.reshape(1, D2))

    zero_init = jnp.zeros((N, D), jnp.float32)
    hpart = _scatter_pass(msg, dst, zero_init)

    out = pl.pallas_call(
        _final_body,
        out_shape=jax.ShapeDtypeStruct((N, D), jnp.float32),
    )(node_feats, hpart, gamma_n.reshape(1, D), beta_n.reshape(1, D))

    return out


# speedup vs baseline: 1.2527x; 1.2527x over previous
import functools

import jax
import jax.numpy as jnp
import numpy as np
from jax import lax
from jax.experimental import pallas as pl
from jax.experimental.pallas import tpu as pltpu
from jax.experimental.pallas import tpu_sc as plsc

N = 10000
E = 320000
D = 128
D2 = 256
EPS = 1e-5

NC = 2
NS = 16
NW = NC * NS
EPW = E // NW
GC = 80
N_GCHUNK = EPW // GC
NG = D2 // 32
DW = D2 // 2

E_PER_CORE = E // NC
EPT = E_PER_CORE // NS
SC2 = 80
N_SCHUNK = EPT // SC2


def _pack_halves(x):
    half = x.shape[-1] // 2
    lo = jax.lax.bitcast_convert_type(
        x[:, :half].astype(jnp.bfloat16), jnp.uint16
    ).astype(jnp.int32)
    hi = jax.lax.bitcast_convert_type(
        x[:, half:].astype(jnp.bfloat16), jnp.uint16
    ).astype(jnp.int32)
    return lax.bitwise_or(lo, lax.shift_left(hi, 16))


def _node_proj_body(nf_ref, w_ref, b_ref, hs_ref, hd_ref):
    nf = nf_ref[...]
    w = w_ref[...]
    b = b_ref[...]
    hs_ref[...] = _pack_halves(
        jnp.dot(nf, w[:, :D2], preferred_element_type=jnp.float32) + b[:, :D2]
    )
    hd_ref[...] = _pack_halves(
        jnp.dot(nf, w[:, D2:], preferred_element_type=jnp.float32) + b[:, D2:]
    )


def _edge_proj_body(f_ref, w_ref, b_ref, out_ref):
    out_ref[...] = _pack_halves(
        jnp.dot(f_ref[...], w_ref[...], preferred_element_type=jnp.float32)
        + b_ref[...]
    )


def _stats_body(m_ref, out_ref):
    i = pl.program_id(0)
    x = m_ref[...]
    st = jnp.concatenate(
        [jnp.sum(x, axis=0, keepdims=True),
         jnp.sum(x * x, axis=0, keepdims=True)],
        axis=0,
    )

    @pl.when(i == 0)
    def _():
        out_ref[...] = st

    @pl.when(i > 0)
    def _():
        out_ref[...] = out_ref[...] + st


def _gate_body(m_ref, stats_ref, gm_ref, bm_ref, msg_ref):
    stats = stats_ref[...]
    mean = stats[0] / E
    var = stats[1] / E - mean * mean
    rstd = lax.rsqrt(var + EPS)
    scale = rstd * gm_ref[0]
    shift = bm_ref[0] - mean * scale
    mhat = m_ref[...] * scale + shift
    h_f = mhat[:, :D]
    h_s = mhat[:, D:]
    msg_ref[...] = jax.nn.sigmoid(h_f) * jax.nn.softplus(h_s)


def _final_body(nf_ref, hp_ref, gn_ref, bn_ref, out_ref):
    h = hp_ref[0] + hp_ref[1]
    mean = jnp.mean(h, axis=0, keepdims=True)
    var = jnp.mean((h - mean) ** 2, axis=0, keepdims=True)
    rstd = lax.rsqrt(var + EPS)
    hn = (h - mean) * rstd * gn_ref[0] + bn_ref[0]
    out_ref[...] = jax.nn.softplus(nf_ref[...] + hn)


def _sc_mesh():
    return plsc.VectorSubcoreMesh(
        core_axis_name="c", subcore_axis_name="s", num_cores=NC, num_subcores=NS
    )


def _gather_pass(hs32, hd32, src, dst, ep32):
    @functools.partial(
        pl.kernel,
        out_type=jax.ShapeDtypeStruct((E, D2), jnp.float32),
        mesh=_sc_mesh(),
        scratch_types=[
            pltpu.VMEM((2, GC), jnp.int32),
            pltpu.VMEM((2, GC), jnp.int32),
            pltpu.VMEM((GC, DW), jnp.int32),
            pltpu.VMEM((GC, DW), jnp.int32),
            pltpu.VMEM((GC, DW), jnp.int32),
            pltpu.VMEM((GC, D2), jnp.float32),
            pltpu.VMEM((GC, DW), jnp.int32),
            pltpu.VMEM((GC, DW), jnp.int32),
            pltpu.VMEM((GC, DW), jnp.int32),
            pltpu.VMEM((GC, D2), jnp.float32),
        ]
        + [pltpu.SemaphoreType.DMA] * 8,
    )
    def k(hs_hbm, hd_hbm, src_hbm, dst_hbm, ep_hbm, m_hbm,
          ia, ib, ra0, rb0, rc0, rm0, ra1, rb1, rc1, rm1,
          sa0, sb0, sc0, sw0, sa1, sb1, sc1, sw1):
        cid = lax.axis_index("c")
        sid = lax.axis_index("s")
        base0 = (sid * NC + cid) * EPW

        bufs = ((ra0, rb0, rc0, rm0, sa0, sb0, sc0, sw0),
                (ra1, rb1, rc1, rm1, sa1, sb1, sc1, sw1))

        def issue(ch, b, drain):
            ra, rb, rc, rm, sa, sb, sc_, sw = bufs[b]
            if drain is not None:
                @pl.when(drain)
                def _():
                    pltpu.make_async_copy(
                        rm, m_hbm.at[pl.ds(pl.multiple_of(base0, 8), GC)], sw
                    ).wait()
            cb = pl.multiple_of(base0 + ch * GC, 8)
            pltpu.sync_copy(src_hbm.at[pl.ds(cb, GC)], ia.at[b])
            pltpu.sync_copy(dst_hbm.at[pl.ds(cb, GC)], ib.at[b])
            pltpu.async_copy(hs_hbm.at[ia.at[b]], ra, sa)
            pltpu.async_copy(hd_hbm.at[ib.at[b]], rb, sb)
            pltpu.async_copy(ep_hbm.at[pl.ds(cb, GC)], rc, sc_)

        hi_mask = jnp.full((16,), -65536, jnp.int32)

        def widen(u):
            ev = lax.bitcast_convert_type(lax.shift_left(u, 16), jnp.float32)
            od = lax.bitcast_convert_type(
                lax.bitwise_and(u, hi_mask), jnp.float32
            )
            return ev, od

        def compute(ch, b):
            ra, rb, rc, rm, sa, sb, sc_, sw = bufs[b]
            lin = ep_hbm.at[pl.ds(pl.multiple_of(base0, 8), GC)]
            pltpu.make_async_copy(lin, ra, sa).wait()
            pltpu.make_async_copy(lin, rb, sb).wait()
            pltpu.make_async_copy(lin, rc, sc_).wait()

            @plsc.parallel_loop(0, GC, 1, unroll=4)
            def _loop(r):
                for g in range(NG):
                    slw = pl.ds(g * 16, 16)
                    ae, ao = widen(ra[r, slw])
                    be, bo = widen(rb[r, slw])
                    ce, co = widen(rc[r, slw])
                    ve = ae + be + ce
                    vo = ao + bo + co
                    rm[r, pl.ds(g * 16, 16)] = ve
                    rm[r, pl.ds(D + g * 16, 16)] = vo

            cb = pl.multiple_of(base0 + ch * GC, 8)
            pltpu.async_copy(rm, m_hbm.at[pl.ds(cb, GC)], sw)

        issue(0, 0, drain=None)

        def body(i, carry):
            @pl.when(i % 2 == 0)
            def _():
                issue(i + 1, 1, drain=i >= 1)
                compute(i, 0)

            @pl.when(i % 2 == 1)
            def _():
                issue(i + 1, 0, drain=i >= 1)
                compute(i, 1)

            return carry

        lax.fori_loop(0, N_GCHUNK - 1, body, 0)
        compute(N_GCHUNK - 1, (N_GCHUNK - 1) % 2)

        for b in (0, 1):
            rm = bufs[b][3]
            sw = bufs[b][7]
            pltpu.make_async_copy(
                rm, m_hbm.at[pl.ds(pl.multiple_of(base0, 8), GC)], sw
            ).wait()

    return k(hs32, hd32, src, dst, ep32)


def _scatter_pass(msg, dst, zero_init):
    @functools.partial(
        pl.kernel,
        out_type=jax.ShapeDtypeStruct((NC, N, D), jnp.float32),
        mesh=_sc_mesh(),
        scratch_types=[
            pltpu.VMEM((SC2,), jnp.int32),
            pltpu.VMEM((SC2, D), jnp.float32),
            pltpu.VMEM_SHARED((N, D), jnp.float32),
        ],
    )
    def k(msg_hbm, dst_hbm, zero_hbm, out_hbm, idx_v, buf, acc_sh):
        cid = lax.axis_index("c")
        sid = lax.axis_index("s")

        @pl.when(sid == 0)
        def _():
            pltpu.sync_copy(zero_hbm, acc_sh)

        plsc.subcore_barrier()

        base0 = cid * E_PER_CORE + sid * EPT

        def chunk_body(ch, carry):
            cb = pl.multiple_of(base0 + ch * SC2, 8)
            pltpu.sync_copy(dst_hbm.at[pl.ds(cb, SC2)], idx_v)
            pltpu.sync_copy(msg_hbm.at[pl.ds(cb, SC2)], buf)
            pltpu.sync_copy(buf, acc_sh.at[idx_v], add=True)
            return carry

        lax.fori_loop(0, N_SCHUNK, chunk_body, 0)
        plsc.subcore_barrier()
        rb = sid * 624

        @pl.when(sid < NS - 1)
        def _():
            pltpu.sync_copy(
                acc_sh.at[pl.ds(rb, 624)], out_hbm.at[cid, pl.ds(rb, 624)]
            )

        @pl.when(sid == NS - 1)
        def _():
            pltpu.sync_copy(
                acc_sh.at[pl.ds(15 * 624, N - 15 * 624)],
                out_hbm.at[cid, pl.ds(15 * 624, N - 15 * 624)],
            )

    return k(msg, dst, zero_init)


def kernel(node_feats, edge_index, edge_feats, W_src, b_src, W_dst, b_dst,
           W_edge, b_edge, gamma_m, beta_m, gamma_n, beta_n):
    src = edge_index[0].astype(jnp.int32)
    dst = edge_index[1].astype(jnp.int32)

    w_cat = jnp.concatenate([W_src.T, W_dst.T], axis=1)
    b_cat = jnp.concatenate([b_src, b_dst]).reshape(1, 2 * D2)

    hs32, hd32 = pl.pallas_call(
        _node_proj_body,
        out_shape=[
            jax.ShapeDtypeStruct((N, DW), jnp.int32),
            jax.ShapeDtypeStruct((N, DW), jnp.int32),
        ],
    )(node_feats, w_cat, b_cat)

    EB = 4000
    ep32 = pl.pallas_call(
        _edge_proj_body,
        grid=(E // EB,),
        in_specs=[
            pl.BlockSpec((EB, 16), lambda i: (i, 0)),
            pl.BlockSpec((16, D2), lambda i: (0, 0)),
            pl.BlockSpec((1, D2), lambda i: (0, 0)),
        ],
        out_specs=pl.BlockSpec((EB, DW), lambda i: (i, 0)),
        out_shape=jax.ShapeDtypeStruct((E, DW), jnp.int32),
    )(edge_feats, W_edge.T, b_edge.reshape(1, D2))

    m = _gather_pass(hs32, hd32, src, dst, ep32)

    stats = pl.pallas_call(
        _stats_body,
        grid=(E // EB,),
        in_specs=[pl.BlockSpec((EB, D2), lambda i: (i, 0))],
        out_specs=pl.BlockSpec((2, D2), lambda i: (0, 0)),
        out_shape=jax.ShapeDtypeStruct((2, D2), jnp.float32),
    )(m)

    msg = pl.pallas_call(
        _gate_body,
        grid=(E // EB,),
        in_specs=[
            pl.BlockSpec((EB, D2), lambda i: (i, 0)),
            pl.BlockSpec((2, D2), lambda i: (0, 0)),
            pl.BlockSpec((1, D2), lambda i: (0, 0)),
            pl.BlockSpec((1, D2), lambda i: (0, 0)),
        ],
        out_specs=pl.BlockSpec((EB, D), lambda i: (i, 0)),
        out_shape=jax.ShapeDtypeStruct((E, D), jnp.float32),
    )(m, stats, gamma_m.reshape(1, D2), beta_m.reshape(1, D2))

    zero_init = jnp.zeros((N, D), jnp.float32)
    hpart = _scatter_pass(msg, dst, zero_init)

    out = pl.pallas_call(
        _final_body,
        out_shape=jax.ShapeDtypeStruct((N, D), jnp.float32),
    )(node_feats, hpart, gamma_n.reshape(1, D), beta_n.reshape(1, D))

    return out


# speedup vs baseline: 1.5504x; 1.2377x over previous
import functools

import jax
import jax.numpy as jnp
import numpy as np
from jax import lax
from jax.experimental import pallas as pl
from jax.experimental.pallas import tpu as pltpu
from jax.experimental.pallas import tpu_sc as plsc

N = 10000
E = 320000
D = 128
D2 = 256
EPS = 1e-5

NC = 2
NS = 16
NW = NC * NS
EPW = E // NW
GC = 80
N_GCHUNK = EPW // GC
NG = D2 // 32
DW = D2 // 2

E_PER_CORE = E // NC
EPT = E_PER_CORE // NS
SC2 = 80
N_SCHUNK = EPT // SC2


def _pack_halves(x):
    half = x.shape[-1] // 2
    lo = jax.lax.bitcast_convert_type(
        x[:, :half].astype(jnp.bfloat16), jnp.uint16
    ).astype(jnp.int32)
    hi = jax.lax.bitcast_convert_type(
        x[:, half:].astype(jnp.bfloat16), jnp.uint16
    ).astype(jnp.int32)
    return lax.bitwise_or(lo, lax.shift_left(hi, 16))


def _node_proj_body(nf_ref, w_ref, b_ref, hs_ref, hd_ref):
    nf = nf_ref[...]
    w = w_ref[...]
    b = b_ref[...]
    hs_ref[...] = _pack_halves(
        jnp.dot(nf, w[:, :D2], preferred_element_type=jnp.float32) + b[:, :D2]
    )
    hd_ref[...] = _pack_halves(
        jnp.dot(nf, w[:, D2:], preferred_element_type=jnp.float32) + b[:, D2:]
    )


def _edge_proj_body(f_ref, w_ref, b_ref, out_ref):
    out_ref[...] = _pack_halves(
        jnp.dot(f_ref[...], w_ref[...], preferred_element_type=jnp.float32)
        + b_ref[...]
    )


def _stats_body(m_ref, out_ref):
    i = pl.program_id(0)
    x = m_ref[...]
    st = jnp.concatenate(
        [jnp.sum(x, axis=0, keepdims=True),
         jnp.sum(x * x, axis=0, keepdims=True)],
        axis=0,
    )

    @pl.when(i == 0)
    def _():
        out_ref[...] = st

    @pl.when(i > 0)
    def _():
        out_ref[...] = out_ref[...] + st


def _gate_body(m_ref, stats_ref, gm_ref, bm_ref, msg_ref):
    stats = stats_ref[...]
    mean = stats[0] / E
    var = stats[1] / E - mean * mean
    rstd = lax.rsqrt(var + EPS)
    scale = rstd * gm_ref[0]
    shift = bm_ref[0] - mean * scale
    mhat = m_ref[...] * scale + shift
    h_f = mhat[:, :D]
    h_s = mhat[:, D:]
    msg_ref[...] = jax.nn.sigmoid(h_f) * jax.nn.softplus(h_s)


def _final_body(nf_ref, hp_ref, gn_ref, bn_ref, out_ref):
    h = hp_ref[0] + hp_ref[1]
    mean = jnp.mean(h, axis=0, keepdims=True)
    var = jnp.mean((h - mean) ** 2, axis=0, keepdims=True)
    rstd = lax.rsqrt(var + EPS)
    hn = (h - mean) * rstd * gn_ref[0] + bn_ref[0]
    out_ref[...] = jax.nn.softplus(nf_ref[...] + hn)


def _sc_mesh():
    return plsc.VectorSubcoreMesh(
        core_axis_name="c", subcore_axis_name="s", num_cores=NC, num_subcores=NS
    )


def _gather_pass(hs32, hd32, src, dst, ep32):
    @functools.partial(
        pl.kernel,
        out_type=jax.ShapeDtypeStruct((E, D2), jnp.float32),
        mesh=_sc_mesh(),
        scratch_types=[
            pltpu.VMEM((EPW,), jnp.int32),
            pltpu.VMEM((EPW,), jnp.int32),
            pltpu.VMEM((GC, DW), jnp.int32),
            pltpu.VMEM((GC, DW), jnp.int32),
            pltpu.VMEM((GC, DW), jnp.int32),
            pltpu.VMEM((GC, D2), jnp.float32),
            pltpu.VMEM((GC, DW), jnp.int32),
            pltpu.VMEM((GC, DW), jnp.int32),
            pltpu.VMEM((GC, DW), jnp.int32),
            pltpu.VMEM((GC, D2), jnp.float32),
        ]
        + [pltpu.SemaphoreType.DMA] * 8,
    )
    def k(hs_hbm, hd_hbm, src_hbm, dst_hbm, ep_hbm, m_hbm,
          ia, ib, ra0, rb0, rc0, rm0, ra1, rb1, rc1, rm1,
          sa0, sb0, sc0, sw0, sa1, sb1, sc1, sw1):
        cid = lax.axis_index("c")
        sid = lax.axis_index("s")
        base0 = (sid * NC + cid) * EPW

        bufs = ((ra0, rb0, rc0, rm0, sa0, sb0, sc0, sw0),
                (ra1, rb1, rc1, rm1, sa1, sb1, sc1, sw1))

        pltpu.sync_copy(src_hbm.at[pl.ds(pl.multiple_of(base0, 8), EPW)], ia)
        pltpu.sync_copy(dst_hbm.at[pl.ds(pl.multiple_of(base0, 8), EPW)], ib)

        def issue(ch, b, drain):
            ra, rb, rc, rm, sa, sb, sc_, sw = bufs[b]
            if drain is not None:
                @pl.when(drain)
                def _():
                    pltpu.make_async_copy(
                        rm, m_hbm.at[pl.ds(pl.multiple_of(base0, 8), GC)], sw
                    ).wait()
            cb = pl.multiple_of(base0 + ch * GC, 8)
            co = pl.multiple_of(ch * GC, 8)
            pltpu.async_copy(hs_hbm.at[ia.at[pl.ds(co, GC)]], ra, sa)
            pltpu.async_copy(hd_hbm.at[ib.at[pl.ds(co, GC)]], rb, sb)
            pltpu.async_copy(ep_hbm.at[pl.ds(cb, GC)], rc, sc_)

        hi_mask = jnp.full((16,), -65536, jnp.int32)

        def widen(u):
            ev = lax.bitcast_convert_type(lax.shift_left(u, 16), jnp.float32)
            od = lax.bitcast_convert_type(
                lax.bitwise_and(u, hi_mask), jnp.float32
            )
            return ev, od

        def compute(ch, b):
            ra, rb, rc, rm, sa, sb, sc_, sw = bufs[b]
            lin = ep_hbm.at[pl.ds(pl.multiple_of(base0, 8), GC)]
            pltpu.make_async_copy(lin, ra, sa).wait()
            pltpu.make_async_copy(lin, rb, sb).wait()
            pltpu.make_async_copy(lin, rc, sc_).wait()

            @plsc.parallel_loop(0, GC, 1, unroll=4)
            def _loop(r):
                for g in range(NG):
                    slw = pl.ds(g * 16, 16)
                    ae, ao = widen(ra[r, slw])
                    be, bo = widen(rb[r, slw])
                    ce, co = widen(rc[r, slw])
                    ve = ae + be + ce
                    vo = ao + bo + co
                    rm[r, pl.ds(g * 16, 16)] = ve
                    rm[r, pl.ds(D + g * 16, 16)] = vo

            cb = pl.multiple_of(base0 + ch * GC, 8)
            pltpu.async_copy(rm, m_hbm.at[pl.ds(cb, GC)], sw)

        issue(0, 0, drain=None)

        def body(i, carry):
            @pl.when(i % 2 == 0)
            def _():
                issue(i + 1, 1, drain=i >= 1)
                compute(i, 0)

            @pl.when(i % 2 == 1)
            def _():
                issue(i + 1, 0, drain=i >= 1)
                compute(i, 1)

            return carry

        lax.fori_loop(0, N_GCHUNK - 1, body, 0)
        compute(N_GCHUNK - 1, (N_GCHUNK - 1) % 2)

        for b in (0, 1):
            rm = bufs[b][3]
            sw = bufs[b][7]
            pltpu.make_async_copy(
                rm, m_hbm.at[pl.ds(pl.multiple_of(base0, 8), GC)], sw
            ).wait()

    return k(hs32, hd32, src, dst, ep32)


def _scatter_pass(msg, dst, zero_init):
    @functools.partial(
        pl.kernel,
        out_type=jax.ShapeDtypeStruct((NC, N, D), jnp.float32),
        mesh=_sc_mesh(),
        scratch_types=[
            pltpu.VMEM((EPT,), jnp.int32),
            pltpu.VMEM((SC2, D), jnp.float32),
            pltpu.VMEM((SC2, D), jnp.float32),
            pltpu.VMEM_SHARED((N, D), jnp.float32),
            pltpu.SemaphoreType.DMA,
            pltpu.SemaphoreType.DMA,
        ],
    )
    def k(msg_hbm, dst_hbm, zero_hbm, out_hbm, idx_v, buf0, buf1, acc_sh,
          sm0, sm1):
        cid = lax.axis_index("c")
        sid = lax.axis_index("s")

        base0 = cid * E_PER_CORE + sid * EPT
        pltpu.sync_copy(dst_hbm.at[pl.ds(pl.multiple_of(base0, 8), EPT)], idx_v)

        @pl.when(sid == 0)
        def _():
            pltpu.sync_copy(zero_hbm, acc_sh)

        plsc.subcore_barrier()

        bufs = ((buf0, sm0), (buf1, sm1))

        def load(ch, b):
            buf, sm = bufs[b]
            cb = pl.multiple_of(base0 + ch * SC2, 8)
            pltpu.async_copy(msg_hbm.at[pl.ds(cb, SC2)], buf, sm)

        def scat(ch, b):
            buf, sm = bufs[b]
            lin = msg_hbm.at[pl.ds(pl.multiple_of(base0, 8), SC2)]
            pltpu.make_async_copy(lin, buf, sm).wait()
            co = pl.multiple_of(ch * SC2, 8)
            pltpu.sync_copy(buf, acc_sh.at[idx_v.at[pl.ds(co, SC2)]], add=True)

        load(0, 0)

        def chunk_body(i, carry):
            @pl.when(i % 2 == 0)
            def _():
                load(i + 1, 1)
                scat(i, 0)

            @pl.when(i % 2 == 1)
            def _():
                load(i + 1, 0)
                scat(i, 1)

            return carry

        lax.fori_loop(0, N_SCHUNK - 1, chunk_body, 0)
        scat(N_SCHUNK - 1, (N_SCHUNK - 1) % 2)

        plsc.subcore_barrier()
        rb = sid * 624

        @pl.when(sid < NS - 1)
        def _():
            pltpu.sync_copy(
                acc_sh.at[pl.ds(rb, 624)], out_hbm.at[cid, pl.ds(rb, 624)]
            )

        @pl.when(sid == NS - 1)
        def _():
            pltpu.sync_copy(
                acc_sh.at[pl.ds(15 * 624, N - 15 * 624)],
                out_hbm.at[cid, pl.ds(15 * 624, N - 15 * 624)],
            )

    return k(msg, dst, zero_init)


def kernel(node_feats, edge_index, edge_feats, W_src, b_src, W_dst, b_dst,
           W_edge, b_edge, gamma_m, beta_m, gamma_n, beta_n):
    src = edge_index[0].astype(jnp.int32)
    dst = edge_index[1].astype(jnp.int32)

    w_cat = jnp.concatenate([W_src.T, W_dst.T], axis=1)
    b_cat = jnp.concatenate([b_src, b_dst]).reshape(1, 2 * D2)

    hs32, hd32 = pl.pallas_call(
        _node_proj_body,
        out_shape=[
            jax.ShapeDtypeStruct((N, DW), jnp.int32),
            jax.ShapeDtypeStruct((N, DW), jnp.int32),
        ],
    )(node_feats, w_cat, b_cat)

    EB = 4000
    ep32 = pl.pallas_call(
        _edge_proj_body,
        grid=(E // EB,),
        in_specs=[
            pl.BlockSpec((EB, 16), lambda i: (i, 0)),
            pl.BlockSpec((16, D2), lambda i: (0, 0)),
            pl.BlockSpec((1, D2), lambda i: (0, 0)),
        ],
        out_specs=pl.BlockSpec((EB, DW), lambda i: (i, 0)),
        out_shape=jax.ShapeDtypeStruct((E, DW), jnp.int32),
    )(edge_feats, W_edge.T, b_edge.reshape(1, D2))

    m = _gather_pass(hs32, hd32, src, dst, ep32)

    stats = pl.pallas_call(
        _stats_body,
        grid=(E // EB,),
        in_specs=[pl.BlockSpec((EB, D2), lambda i: (i, 0))],
        out_specs=pl.BlockSpec((2, D2), lambda i: (0, 0)),
        out_shape=jax.ShapeDtypeStruct((2, D2), jnp.float32),
    )(m)

    msg = pl.pallas_call(
        _gate_body,
        grid=(E // EB,),
        in_specs=[
            pl.BlockSpec((EB, D2), lambda i: (i, 0)),
            pl.BlockSpec((2, D2), lambda i: (0, 0)),
            pl.BlockSpec((1, D2), lambda i: (0, 0)),
            pl.BlockSpec((1, D2), lambda i: (0, 0)),
        ],
        out_specs=pl.BlockSpec((EB, D), lambda i: (i, 0)),
        out_shape=jax.ShapeDtypeStruct((E, D), jnp.float32),
    )(m, stats, gamma_m.reshape(1, D2), beta_m.reshape(1, D2))

    zero_init = jnp.zeros((N, D), jnp.float32)
    hpart = _scatter_pass(msg, dst, zero_init)

    out = pl.pallas_call(
        _final_body,
        out_shape=jax.ShapeDtypeStruct((N, D), jnp.float32),
    )(node_feats, hpart, gamma_n.reshape(1, D), beta_n.reshape(1, D))

    return out


# speedup vs baseline: 1.5513x; 1.0005x over previous
import functools

import jax
import jax.numpy as jnp
import numpy as np
from jax import lax
from jax.experimental import pallas as pl
from jax.experimental.pallas import tpu as pltpu
from jax.experimental.pallas import tpu_sc as plsc

N = 10000
E = 320000
D = 128
D2 = 256
EPS = 1e-5

NC = 2
NS = 16
NW = NC * NS
EPW = E // NW
GC = 80
N_GCHUNK = EPW // GC
NG = D2 // 32
DW = D2 // 2

E_PER_CORE = E // NC
EPT = E_PER_CORE // NS
SC2 = 80
N_SCHUNK = EPT // SC2


def _pack_halves(x):
    half = x.shape[-1] // 2
    lo = jax.lax.bitcast_convert_type(
        x[:, :half].astype(jnp.bfloat16), jnp.uint16
    ).astype(jnp.int32)
    hi = jax.lax.bitcast_convert_type(
        x[:, half:].astype(jnp.bfloat16), jnp.uint16
    ).astype(jnp.int32)
    return lax.bitwise_or(lo, lax.shift_left(hi, 16))


def _node_proj_body(nf_ref, w_ref, b_ref, hs_ref, hd_ref):
    nf = nf_ref[...]
    w = w_ref[...]
    b = b_ref[...]
    hs_ref[...] = _pack_halves(
        jnp.dot(nf, w[:, :D2], preferred_element_type=jnp.float32) + b[:, :D2]
    )
    hd_ref[...] = _pack_halves(
        jnp.dot(nf, w[:, D2:], preferred_element_type=jnp.float32) + b[:, D2:]
    )


def _edge_proj_body(f_ref, w_ref, b_ref, out_ref):
    out_ref[...] = _pack_halves(
        jnp.dot(f_ref[...], w_ref[...], preferred_element_type=jnp.float32)
        + b_ref[...]
    )


def _stats_body(m_ref, out_ref):
    i = pl.program_id(0)
    x = m_ref[...]
    st = jnp.concatenate(
        [jnp.sum(x, axis=0, keepdims=True),
         jnp.sum(x * x, axis=0, keepdims=True)],
        axis=0,
    )

    @pl.when(i == 0)
    def _():
        out_ref[...] = st

    @pl.when(i > 0)
    def _():
        out_ref[...] = out_ref[...] + st


def _gate_body(m_ref, stats_ref, gm_ref, bm_ref, msg_ref):
    stats = stats_ref[...]
    mean = stats[0] / E
    var = stats[1] / E - mean * mean
    rstd = lax.rsqrt(var + EPS)
    scale = rstd * gm_ref[0]
    shift = bm_ref[0] - mean * scale
    mhat = m_ref[...] * scale + shift
    h_f = mhat[:, :D]
    h_s = mhat[:, D:]
    msg_ref[...] = jax.nn.sigmoid(h_f) * jax.nn.softplus(h_s)


def _final_body(nf_ref, hp_ref, gn_ref, bn_ref, out_ref):
    h = hp_ref[0] + hp_ref[1]
    mean = jnp.mean(h, axis=0, keepdims=True)
    var = jnp.mean((h - mean) ** 2, axis=0, keepdims=True)
    rstd = lax.rsqrt(var + EPS)
    hn = (h - mean) * rstd * gn_ref[0] + bn_ref[0]
    out_ref[...] = jax.nn.softplus(nf_ref[...] + hn)


def _sc_mesh():
    return plsc.VectorSubcoreMesh(
        core_axis_name="c", subcore_axis_name="s", num_cores=NC, num_subcores=NS
    )


def _gather_pass(hs32, hd32, src, dst, ep32):
    @functools.partial(
        pl.kernel,
        out_type=jax.ShapeDtypeStruct((E, D2), jnp.float32),
        mesh=_sc_mesh(),
        scratch_types=[
            pltpu.VMEM((EPW,), jnp.int32),
            pltpu.VMEM((EPW,), jnp.int32),
            pltpu.VMEM((GC, DW), jnp.int32),
            pltpu.VMEM((GC, DW), jnp.int32),
            pltpu.VMEM((GC, DW), jnp.int32),
            pltpu.VMEM((GC, D2), jnp.float32),
            pltpu.VMEM((GC, DW), jnp.int32),
            pltpu.VMEM((GC, DW), jnp.int32),
            pltpu.VMEM((GC, DW), jnp.int32),
            pltpu.VMEM((GC, D2), jnp.float32),
        ]
        + [pltpu.SemaphoreType.DMA] * 8,
    )
    def k(hs_hbm, hd_hbm, src_hbm, dst_hbm, ep_hbm, m_hbm,
          ia, ib, ra0, rb0, rc0, rm0, ra1, rb1, rc1, rm1,
          sa0, sb0, sc0, sw0, sa1, sb1, sc1, sw1):
        cid = lax.axis_index("c")
        sid = lax.axis_index("s")
        base0 = (sid * NC + cid) * EPW

        bufs = ((ra0, rb0, rc0, rm0, sa0, sb0, sc0, sw0),
                (ra1, rb1, rc1, rm1, sa1, sb1, sc1, sw1))

        pltpu.sync_copy(src_hbm.at[pl.ds(pl.multiple_of(base0, 8), EPW)], ia)
        pltpu.sync_copy(dst_hbm.at[pl.ds(pl.multiple_of(base0, 8), EPW)], ib)

        def issue(ch, b, drain):
            ra, rb, rc, rm, sa, sb, sc_, sw = bufs[b]
            if drain is not None:
                @pl.when(drain)
                def _():
                    pltpu.make_async_copy(
                        rm, m_hbm.at[pl.ds(pl.multiple_of(base0, 8), GC)], sw
                    ).wait()
            cb = pl.multiple_of(base0 + ch * GC, 8)
            co = pl.multiple_of(ch * GC, 8)
            pltpu.async_copy(hs_hbm.at[ia.at[pl.ds(co, GC)]], ra, sa)
            pltpu.async_copy(hd_hbm.at[ib.at[pl.ds(co, GC)]], rb, sb)
            pltpu.async_copy(ep_hbm.at[pl.ds(cb, GC)], rc, sc_)

        hi_mask = jnp.full((16,), -65536, jnp.int32)

        def widen(u):
            ev = lax.bitcast_convert_type(lax.shift_left(u, 16), jnp.float32)
            od = lax.bitcast_convert_type(
                lax.bitwise_and(u, hi_mask), jnp.float32
            )
            return ev, od

        def compute(ch, b):
            ra, rb, rc, rm, sa, sb, sc_, sw = bufs[b]
            lin = ep_hbm.at[pl.ds(pl.multiple_of(base0, 8), GC)]
            pltpu.make_async_copy(lin, ra, sa).wait()
            pltpu.make_async_copy(lin, rb, sb).wait()
            pltpu.make_async_copy(lin, rc, sc_).wait()

            @plsc.parallel_loop(0, GC, 1, unroll=4)
            def _loop(r):
                for g in range(NG):
                    slw = pl.ds(g * 16, 16)
                    ae, ao = widen(ra[r, slw])
                    be, bo = widen(rb[r, slw])
                    ce, co = widen(rc[r, slw])
                    ve = ae + be + ce
                    vo = ao + bo + co
                    rm[r, pl.ds(g * 16, 16)] = ve
                    rm[r, pl.ds(D + g * 16, 16)] = vo

            cb = pl.multiple_of(base0 + ch * GC, 8)
            pltpu.async_copy(rm, m_hbm.at[pl.ds(cb, GC)], sw)

        issue(0, 0, drain=None)

        def body(i, carry):
            @pl.when(i % 2 == 0)
            def _():
                issue(i + 1, 1, drain=i >= 1)
                compute(i, 0)

            @pl.when(i % 2 == 1)
            def _():
                issue(i + 1, 0, drain=i >= 1)
                compute(i, 1)

            return carry

        lax.fori_loop(0, N_GCHUNK - 1, body, 0)
        compute(N_GCHUNK - 1, (N_GCHUNK - 1) % 2)

        for b in (0, 1):
            rm = bufs[b][3]
            sw = bufs[b][7]
            pltpu.make_async_copy(
                rm, m_hbm.at[pl.ds(pl.multiple_of(base0, 8), GC)], sw
            ).wait()

    return k(hs32, hd32, src, dst, ep32)


def _scatter_pass(msg, dst, zero_init):
    @functools.partial(
        pl.kernel,
        out_type=jax.ShapeDtypeStruct((NC, N, D), jnp.float32),
        mesh=_sc_mesh(),
        scratch_types=[
            pltpu.VMEM((SC2,), jnp.int32),
            pltpu.VMEM((SC2,), jnp.int32),
            pltpu.VMEM((SC2, D), jnp.float32),
            pltpu.VMEM((SC2, D), jnp.float32),
            pltpu.VMEM_SHARED((N, D), jnp.float32),
            pltpu.SemaphoreType.DMA,
            pltpu.SemaphoreType.DMA,
            pltpu.SemaphoreType.DMA,
            pltpu.SemaphoreType.DMA,
        ],
    )
    def k(msg_hbm, dst_hbm, zero_hbm, out_hbm, idx0, idx1, buf0, buf1, acc_sh,
          sm0, sm1, si0, si1):
        cid = lax.axis_index("c")
        sid = lax.axis_index("s")

        base0 = cid * E_PER_CORE + sid * EPT

        @pl.when(sid == 0)
        def _():
            pltpu.sync_copy(zero_hbm, acc_sh)

        plsc.subcore_barrier()

        bufs = ((idx0, buf0, sm0, si0), (idx1, buf1, sm1, si1))

        def load(ch, b):
            idx, buf, sm, si = bufs[b]
            cb = pl.multiple_of(base0 + ch * SC2, 8)
            pltpu.async_copy(dst_hbm.at[pl.ds(cb, SC2)], idx, si)
            pltpu.async_copy(msg_hbm.at[pl.ds(cb, SC2)], buf, sm)

        def scat(ch, b):
            idx, buf, sm, si = bufs[b]
            lin = msg_hbm.at[pl.ds(pl.multiple_of(base0, 8), SC2)]
            pltpu.make_async_copy(
                dst_hbm.at[pl.ds(pl.multiple_of(base0, 8), SC2)], idx, si
            ).wait()
            pltpu.make_async_copy(lin, buf, sm).wait()
            pltpu.sync_copy(buf, acc_sh.at[idx], add=True)

        load(0, 0)

        def chunk_body(i, carry):
            @pl.when(i % 2 == 0)
            def _():
                load(i + 1, 1)
                scat(i, 0)

            @pl.when(i % 2 == 1)
            def _():
                load(i + 1, 0)
                scat(i, 1)

            return carry

        lax.fori_loop(0, N_SCHUNK - 1, chunk_body, 0)
        scat(N_SCHUNK - 1, (N_SCHUNK - 1) % 2)

        plsc.subcore_barrier()
        rb = sid * 624

        @pl.when(sid < NS - 1)
        def _():
            pltpu.sync_copy(
                acc_sh.at[pl.ds(rb, 624)], out_hbm.at[cid, pl.ds(rb, 624)]
            )

        @pl.when(sid == NS - 1)
        def _():
            pltpu.sync_copy(
                acc_sh.at[pl.ds(15 * 624, N - 15 * 624)],
                out_hbm.at[cid, pl.ds(15 * 624, N - 15 * 624)],
            )

    return k(msg, dst, zero_init)


def kernel(node_feats, edge_index, edge_feats, W_src, b_src, W_dst, b_dst,
           W_edge, b_edge, gamma_m, beta_m, gamma_n, beta_n):
    src = edge_index[0].astype(jnp.int32)
    dst = edge_index[1].astype(jnp.int32)

    w_cat = jnp.concatenate([W_src.T, W_dst.T], axis=1)
    b_cat = jnp.concatenate([b_src, b_dst]).reshape(1, 2 * D2)

    hs32, hd32 = pl.pallas_call(
        _node_proj_body,
        out_shape=[
            jax.ShapeDtypeStruct((N, DW), jnp.int32),
            jax.ShapeDtypeStruct((N, DW), jnp.int32),
        ],
    )(node_feats, w_cat, b_cat)

    EB = 4000
    ep32 = pl.pallas_call(
        _edge_proj_body,
        grid=(E // EB,),
        in_specs=[
            pl.BlockSpec((EB, 16), lambda i: (i, 0)),
            pl.BlockSpec((16, D2), lambda i: (0, 0)),
            pl.BlockSpec((1, D2), lambda i: (0, 0)),
        ],
        out_specs=pl.BlockSpec((EB, DW), lambda i: (i, 0)),
        out_shape=jax.ShapeDtypeStruct((E, DW), jnp.int32),
    )(edge_feats, W_edge.T, b_edge.reshape(1, D2))

    m = _gather_pass(hs32, hd32, src, dst, ep32)

    stats = pl.pallas_call(
        _stats_body,
        grid=(E // EB,),
        in_specs=[pl.BlockSpec((EB, D2), lambda i: (i, 0))],
        out_specs=pl.BlockSpec((2, D2), lambda i: (0, 0)),
        out_shape=jax.ShapeDtypeStruct((2, D2), jnp.float32),
    )(m)

    msg = pl.pallas_call(
        _gate_body,
        grid=(E // EB,),
        in_specs=[
            pl.BlockSpec((EB, D2), lambda i: (i, 0)),
            pl.BlockSpec((2, D2), lambda i: (0, 0)),
            pl.BlockSpec((1, D2), lambda i: (0, 0)),
            pl.BlockSpec((1, D2), lambda i: (0, 0)),
        ],
        out_specs=pl.BlockSpec((EB, D), lambda i: (i, 0)),
        out_shape=jax.ShapeDtypeStruct((E, D), jnp.float32),
    )(m, stats, gamma_m.reshape(1, D2), beta_m.reshape(1, D2))

    zero_init = jnp.zeros((N, D), jnp.float32)
    hpart = _scatter_pass(msg, dst, zero_init)

    out = pl.pallas_call(
        _final_body,
        out_shape=jax.ShapeDtypeStruct((N, D), jnp.float32),
    )(node_feats, hpart, gamma_n.reshape(1, D), beta_n.reshape(1, D))

    return out


# speedup vs baseline: 1.6404x; 1.0574x over previous
import functools

import jax
import jax.numpy as jnp
import numpy as np
from jax import lax
from jax.experimental import pallas as pl
from jax.experimental.pallas import tpu as pltpu
from jax.experimental.pallas import tpu_sc as plsc

N = 10000
E = 320000
D = 128
D2 = 256
EPS = 1e-5

NC = 2
NS = 16
NW = NC * NS
EPW = E // NW
GC = 80
N_GCHUNK = EPW // GC
NG = D2 // 32
DW = D2 // 2

E_PER_CORE = E // NC
EPT = E_PER_CORE // NS
SC2 = 80
N_SCHUNK = EPT // SC2


def _pack_halves(x):
    half = x.shape[-1] // 2
    lo = jax.lax.bitcast_convert_type(
        x[:, :half].astype(jnp.bfloat16), jnp.uint16
    ).astype(jnp.int32)
    hi = jax.lax.bitcast_convert_type(
        x[:, half:].astype(jnp.bfloat16), jnp.uint16
    ).astype(jnp.int32)
    return lax.bitwise_or(lo, lax.shift_left(hi, 16))


def _node_proj_body(nf_ref, w_ref, b_ref, hs_ref, hd_ref):
    nf = nf_ref[...]
    w = w_ref[...]
    b = b_ref[...]
    hs_ref[...] = _pack_halves(
        jnp.dot(nf, w[:, :D2], preferred_element_type=jnp.float32) + b[:, :D2]
    )
    hd_ref[...] = _pack_halves(
        jnp.dot(nf, w[:, D2:], preferred_element_type=jnp.float32) + b[:, D2:]
    )


def _edge_proj_body(f_ref, w_ref, b_ref, out_ref):
    out_ref[...] = _pack_halves(
        jnp.dot(f_ref[...], w_ref[...], preferred_element_type=jnp.float32)
        + b_ref[...]
    )


def _unpack_halves(u):
    lo = lax.bitcast_convert_type(
        lax.convert_element_type(lax.bitwise_and(u, 65535), jnp.uint16),
        jnp.bfloat16,
    ).astype(jnp.float32)
    hi = lax.bitcast_convert_type(
        lax.convert_element_type(
            lax.shift_right_logical(u, 16), jnp.uint16
        ),
        jnp.bfloat16,
    ).astype(jnp.float32)
    return lo, hi


def _stats_body(m_ref, out_ref):
    i = pl.program_id(0)
    lo, hi = _unpack_halves(m_ref[...])
    st = jnp.concatenate(
        [jnp.sum(lo, axis=0, keepdims=True),
         jnp.sum(hi, axis=0, keepdims=True),
         jnp.sum(lo * lo, axis=0, keepdims=True),
         jnp.sum(hi * hi, axis=0, keepdims=True)],
        axis=0,
    )

    @pl.when(i == 0)
    def _():
        out_ref[...] = st

    @pl.when(i > 0)
    def _():
        out_ref[...] = out_ref[...] + st


def _gate_body(m_ref, stats_ref, gm_ref, bm_ref, msg_ref):
    stats = stats_ref[...]
    mean = stats[:2] / E
    var = stats[2:] / E - mean * mean
    rstd = lax.rsqrt(var + EPS)
    scale = rstd * gm_ref[...]
    shift = bm_ref[...] - mean * scale
    lo, hi = _unpack_halves(m_ref[...])
    h_f = lo * scale[0] + shift[0]
    h_s = hi * scale[1] + shift[1]
    msg_ref[...] = jax.nn.sigmoid(h_f) * jax.nn.softplus(h_s)


def _final_body(nf_ref, hp_ref, gn_ref, bn_ref, out_ref):
    h = hp_ref[0] + hp_ref[1]
    mean = jnp.mean(h, axis=0, keepdims=True)
    var = jnp.mean((h - mean) ** 2, axis=0, keepdims=True)
    rstd = lax.rsqrt(var + EPS)
    hn = (h - mean) * rstd * gn_ref[0] + bn_ref[0]
    out_ref[...] = jax.nn.softplus(nf_ref[...] + hn)


def _sc_mesh():
    return plsc.VectorSubcoreMesh(
        core_axis_name="c", subcore_axis_name="s", num_cores=NC, num_subcores=NS
    )


def _gather_pass(hs32, hd32, src, dst, ep32):
    @functools.partial(
        pl.kernel,
        out_type=jax.ShapeDtypeStruct((E, DW), jnp.int32),
        mesh=_sc_mesh(),
        scratch_types=[
            pltpu.VMEM((EPW,), jnp.int32),
            pltpu.VMEM((EPW,), jnp.int32),
            pltpu.VMEM((GC, DW), jnp.int32),
            pltpu.VMEM((GC, DW), jnp.int32),
            pltpu.VMEM((GC, DW), jnp.int32),
            pltpu.VMEM((GC, DW), jnp.int32),
            pltpu.VMEM((GC, DW), jnp.int32),
            pltpu.VMEM((GC, DW), jnp.int32),
            pltpu.VMEM((GC, DW), jnp.int32),
            pltpu.VMEM((GC, DW), jnp.int32),
        ]
        + [pltpu.SemaphoreType.DMA] * 8,
    )
    def k(hs_hbm, hd_hbm, src_hbm, dst_hbm, ep_hbm, m_hbm,
          ia, ib, ra0, rb0, rc0, rm0, ra1, rb1, rc1, rm1,
          sa0, sb0, sc0, sw0, sa1, sb1, sc1, sw1):
        cid = lax.axis_index("c")
        sid = lax.axis_index("s")
        base0 = (sid * NC + cid) * EPW

        bufs = ((ra0, rb0, rc0, rm0, sa0, sb0, sc0, sw0),
                (ra1, rb1, rc1, rm1, sa1, sb1, sc1, sw1))

        pltpu.sync_copy(src_hbm.at[pl.ds(pl.multiple_of(base0, 8), EPW)], ia)
        pltpu.sync_copy(dst_hbm.at[pl.ds(pl.multiple_of(base0, 8), EPW)], ib)

        def issue(ch, b, drain):
            ra, rb, rc, rm, sa, sb, sc_, sw = bufs[b]
            if drain is not None:
                @pl.when(drain)
                def _():
                    pltpu.make_async_copy(
                        rm, m_hbm.at[pl.ds(pl.multiple_of(base0, 8), GC)], sw
                    ).wait()
            cb = pl.multiple_of(base0 + ch * GC, 8)
            co = pl.multiple_of(ch * GC, 8)
            pltpu.async_copy(hs_hbm.at[ia.at[pl.ds(co, GC)]], ra, sa)
            pltpu.async_copy(hd_hbm.at[ib.at[pl.ds(co, GC)]], rb, sb)
            pltpu.async_copy(ep_hbm.at[pl.ds(cb, GC)], rc, sc_)

        hi_mask = jnp.full((16,), -65536, jnp.int32)

        def widen(u):
            ev = lax.bitcast_convert_type(lax.shift_left(u, 16), jnp.float32)
            od = lax.bitcast_convert_type(
                lax.bitwise_and(u, hi_mask), jnp.float32
            )
            return ev, od

        def compute(ch, b):
            ra, rb, rc, rm, sa, sb, sc_, sw = bufs[b]
            lin = ep_hbm.at[pl.ds(pl.multiple_of(base0, 8), GC)]
            pltpu.make_async_copy(lin, ra, sa).wait()
            pltpu.make_async_copy(lin, rb, sb).wait()
            pltpu.make_async_copy(lin, rc, sc_).wait()

            lo_mask = jnp.full((16,), 65535, jnp.int32)

            @plsc.parallel_loop(0, GC, 1, unroll=4)
            def _loop(r):
                for g in range(NG):
                    slw = pl.ds(g * 16, 16)
                    ae, ao = widen(ra[r, slw])
                    be, bo = widen(rb[r, slw])
                    ce, co = widen(rc[r, slw])
                    ve = ae + be + ce
                    vo = ao + bo + co
                    lo = lax.bitwise_and(
                        lax.shift_right_logical(
                            lax.bitcast_convert_type(ve, jnp.int32), 16
                        ),
                        lo_mask,
                    )
                    hi = lax.bitwise_and(
                        lax.bitcast_convert_type(vo, jnp.int32), hi_mask
                    )
                    rm[r, slw] = lax.bitwise_or(lo, hi)

            cb = pl.multiple_of(base0 + ch * GC, 8)
            pltpu.async_copy(rm, m_hbm.at[pl.ds(cb, GC)], sw)

        issue(0, 0, drain=None)

        def body(i, carry):
            @pl.when(i % 2 == 0)
            def _():
                issue(i + 1, 1, drain=i >= 1)
                compute(i, 0)

            @pl.when(i % 2 == 1)
            def _():
                issue(i + 1, 0, drain=i >= 1)
                compute(i, 1)

            return carry

        lax.fori_loop(0, N_GCHUNK - 1, body, 0)
        compute(N_GCHUNK - 1, (N_GCHUNK - 1) % 2)

        for b in (0, 1):
            rm = bufs[b][3]
            sw = bufs[b][7]
            pltpu.make_async_copy(
                rm, m_hbm.at[pl.ds(pl.multiple_of(base0, 8), GC)], sw
            ).wait()

    return k(hs32, hd32, src, dst, ep32)


def _scatter_pass(msg, dst, zero_init):
    @functools.partial(
        pl.kernel,
        out_type=jax.ShapeDtypeStruct((NC, N, D), jnp.float32),
        mesh=_sc_mesh(),
        scratch_types=[
            pltpu.VMEM((SC2,), jnp.int32),
            pltpu.VMEM((SC2,), jnp.int32),
            pltpu.VMEM((SC2, D), jnp.float32),
            pltpu.VMEM((SC2, D), jnp.float32),
            pltpu.VMEM_SHARED((N, D), jnp.float32),
            pltpu.SemaphoreType.DMA,
            pltpu.SemaphoreType.DMA,
            pltpu.SemaphoreType.DMA,
            pltpu.SemaphoreType.DMA,
        ],
    )
    def k(msg_hbm, dst_hbm, zero_hbm, out_hbm, idx0, idx1, buf0, buf1, acc_sh,
          sm0, sm1, si0, si1):
        cid = lax.axis_index("c")
        sid = lax.axis_index("s")

        base0 = cid * E_PER_CORE + sid * EPT

        @pl.when(sid == 0)
        def _():
            pltpu.sync_copy(zero_hbm, acc_sh)

        plsc.subcore_barrier()

        bufs = ((idx0, buf0, sm0, si0), (idx1, buf1, sm1, si1))

        def load(ch, b):
            idx, buf, sm, si = bufs[b]
            cb = pl.multiple_of(base0 + ch * SC2, 8)
            pltpu.async_copy(dst_hbm.at[pl.ds(cb, SC2)], idx, si)
            pltpu.async_copy(msg_hbm.at[pl.ds(cb, SC2)], buf, sm)

        def scat(ch, b):
            idx, buf, sm, si = bufs[b]
            lin = msg_hbm.at[pl.ds(pl.multiple_of(base0, 8), SC2)]
            pltpu.make_async_copy(
                dst_hbm.at[pl.ds(pl.multiple_of(base0, 8), SC2)], idx, si
            ).wait()
            pltpu.make_async_copy(lin, buf, sm).wait()
            pltpu.sync_copy(buf, acc_sh.at[idx], add=True)

        load(0, 0)

        def chunk_body(i, carry):
            @pl.when(i % 2 == 0)
            def _():
                load(i + 1, 1)
                scat(i, 0)

            @pl.when(i % 2 == 1)
            def _():
                load(i + 1, 0)
                scat(i, 1)

            return carry

        lax.fori_loop(0, N_SCHUNK - 1, chunk_body, 0)
        scat(N_SCHUNK - 1, (N_SCHUNK - 1) % 2)

        plsc.subcore_barrier()
        rb = sid * 624

        @pl.when(sid < NS - 1)
        def _():
            pltpu.sync_copy(
                acc_sh.at[pl.ds(rb, 624)], out_hbm.at[cid, pl.ds(rb, 624)]
            )

        @pl.when(sid == NS - 1)
        def _():
            pltpu.sync_copy(
                acc_sh.at[pl.ds(15 * 624, N - 15 * 624)],
                out_hbm.at[cid, pl.ds(15 * 624, N - 15 * 624)],
            )

    return k(msg, dst, zero_init)


def kernel(node_feats, edge_index, edge_feats, W_src, b_src, W_dst, b_dst,
           W_edge, b_edge, gamma_m, beta_m, gamma_n, beta_n):
    src = edge_index[0].astype(jnp.int32)
    dst = edge_index[1].astype(jnp.int32)

    w_cat = jnp.concatenate([W_src.T, W_dst.T], axis=1)
    b_cat = jnp.concatenate([b_src, b_dst]).reshape(1, 2 * D2)

    hs32, hd32 = pl.pallas_call(
        _node_proj_body,
        out_shape=[
            jax.ShapeDtypeStruct((N, DW), jnp.int32),
            jax.ShapeDtypeStruct((N, DW), jnp.int32),
        ],
    )(node_feats, w_cat, b_cat)

    EB = 4000
    ep32 = pl.pallas_call(
        _edge_proj_body,
        grid=(E // EB,),
        in_specs=[
            pl.BlockSpec((EB, 16), lambda i: (i, 0)),
            pl.BlockSpec((16, D2), lambda i: (0, 0)),
            pl.BlockSpec((1, D2), lambda i: (0, 0)),
        ],
        out_specs=pl.BlockSpec((EB, DW), lambda i: (i, 0)),
        out_shape=jax.ShapeDtypeStruct((E, DW), jnp.int32),
    )(edge_feats, W_edge.T, b_edge.reshape(1, D2))

    m = _gather_pass(hs32, hd32, src, dst, ep32)

    stats = pl.pallas_call(
        _stats_body,
        grid=(E // EB,),
        in_specs=[pl.BlockSpec((EB, DW), lambda i: (i, 0))],
        out_specs=pl.BlockSpec((4, D), lambda i: (0, 0)),
        out_shape=jax.ShapeDtypeStruct((4, D), jnp.float32),
    )(m)

    msg = pl.pallas_call(
        _gate_body,
        grid=(E // EB,),
        in_specs=[
            pl.BlockSpec((EB, DW), lambda i: (i, 0)),
            pl.BlockSpec((4, D), lambda i: (0, 0)),
            pl.BlockSpec((2, D), lambda i: (0, 0)),
            pl.BlockSpec((2, D), lambda i: (0, 0)),
        ],
        out_specs=pl.BlockSpec((EB, D), lambda i: (i, 0)),
        out_shape=jax.ShapeDtypeStruct((E, D), jnp.float32),
    )(m, stats, gamma_m.reshape(2, D), beta_m.reshape(2, D))

    zero_init = jnp.zeros((N, D), jnp.float32)
    hpart = _scatter_pass(msg, dst, zero_init)

    out = pl.pallas_call(
        _final_body,
        out_shape=jax.ShapeDtypeStruct((N, D), jnp.float32),
    )(node_feats, hpart, gamma_n.reshape(1, D), beta_n.reshape(1, D))

    return out
